# baseline (speedup 1.0000x reference)
import jax
import jax.numpy as jnp
from jax.experimental import pallas as pl
from jax.experimental.pallas import tpu as pltpu

CAPACITY = 100000
D_MODEL = 512
BATCH = 4096

BLOCK_ROWS = 4096
NB_EMB = BATCH // BLOCK_ROWS
GRID = (CAPACITY + BLOCK_ROWS - 1) // BLOCK_ROWS
LAST = CAPACITY - (GRID - 1) * BLOCK_ROWS


def _body(emb_ref, sig_ref, mem_ref, imp_ref, out_mem, out_imp_ref, sem_m):
    i = pl.program_id(0)
    off = pl.multiple_of(i * BLOCK_ROWS, 8)

    @pl.when(i < NB_EMB)
    def _():
        pltpu.make_async_copy(emb_ref, out_mem.at[pl.ds(0, BATCH)], sem_m).start()
        out_imp_ref[...] = sig_ref[...]
        pltpu.make_async_copy(emb_ref, out_mem.at[pl.ds(0, BATCH)], sem_m).wait()

    @pl.when(jnp.logical_and(i >= NB_EMB, i < GRID - 1))
    def _():
        pltpu.make_async_copy(mem_ref, out_mem.at[pl.ds(off, BLOCK_ROWS)],
                              sem_m).start()
        out_imp_ref[...] = imp_ref[...]
        pltpu.make_async_copy(mem_ref, out_mem.at[pl.ds(off, BLOCK_ROWS)],
                              sem_m).wait()

    @pl.when(i == GRID - 1)
    def _():
        pltpu.make_async_copy(mem_ref.at[pl.ds(0, LAST)],
                              out_mem.at[pl.ds(off, LAST)], sem_m).start()
        out_imp_ref[...] = imp_ref[...]
        pltpu.make_async_copy(mem_ref.at[pl.ds(0, LAST)],
                              out_mem.at[pl.ds(off, LAST)], sem_m).wait()


def kernel(embeddings, loss_signal, memory, importance):
    emb_last = NB_EMB - 1
    out_mem, out_imp = pl.pallas_call(
        _body,
        grid=(GRID,),
        in_specs=[
            pl.BlockSpec((BLOCK_ROWS, D_MODEL), lambda i: (jnp.minimum(i, emb_last), 0)),
            pl.BlockSpec((BLOCK_ROWS,), lambda i: (jnp.minimum(i, emb_last),)),
            pl.BlockSpec((BLOCK_ROWS, D_MODEL), lambda i: (jnp.maximum(i, NB_EMB), 0)),
            pl.BlockSpec((BLOCK_ROWS,), lambda i: (jnp.maximum(i, NB_EMB),)),
        ],
        out_specs=[
            pl.BlockSpec(memory_space=pl.ANY),
            pl.BlockSpec((BLOCK_ROWS,), lambda i: (i,)),
        ],
        out_shape=[
            jax.ShapeDtypeStruct((CAPACITY, D_MODEL), jnp.float32),
            jax.ShapeDtypeStruct((CAPACITY,), jnp.float32),
        ],
        scratch_shapes=[pltpu.SemaphoreType.DMA],
        compiler_params=pltpu.CompilerParams(
            dimension_semantics=("arbitrary",)),
    )(embeddings, loss_signal, memory, importance)

    return out_mem, out_imp


# speedup vs baseline: 1.0202x; 1.0202x over previous
import jax
import jax.numpy as jnp
from jax.experimental import pallas as pl
from jax.experimental.pallas import tpu as pltpu

CAPACITY = 100000
D_MODEL = 512
BATCH = 4096

BLOCK_ROWS = 4096
NB_EMB = BATCH // BLOCK_ROWS
GRID = (CAPACITY + BLOCK_ROWS - 1) // BLOCK_ROWS


def _body(emb_ref, sig_ref, mem_ref, imp_ref, out_mem_ref, out_imp_ref):
    i = pl.program_id(0)

    @pl.when(i < NB_EMB)
    def _():
        out_mem_ref[...] = emb_ref[...]
        out_imp_ref[...] = sig_ref[...]

    @pl.when(i >= NB_EMB)
    def _():
        out_mem_ref[...] = mem_ref[...]
        out_imp_ref[...] = imp_ref[...]


def kernel(embeddings, loss_signal, memory, importance):
    emb_last = NB_EMB - 1
    out_mem, out_imp = pl.pallas_call(
        _body,
        grid=(GRID,),
        in_specs=[
            pl.BlockSpec((BLOCK_ROWS, D_MODEL), lambda i: (jnp.minimum(i, emb_last), 0)),
            pl.BlockSpec((BLOCK_ROWS,), lambda i: (jnp.minimum(i, emb_last),)),
            pl.BlockSpec((BLOCK_ROWS, D_MODEL), lambda i: (jnp.maximum(i, NB_EMB), 0)),
            pl.BlockSpec((BLOCK_ROWS,), lambda i: (jnp.maximum(i, NB_EMB),)),
        ],
        out_specs=[
            pl.BlockSpec((BLOCK_ROWS, D_MODEL), lambda i: (i, 0)),
            pl.BlockSpec((BLOCK_ROWS,), lambda i: (i,)),
        ],
        out_shape=[
            jax.ShapeDtypeStruct((CAPACITY, D_MODEL), jnp.float32),
            jax.ShapeDtypeStruct((CAPACITY,), jnp.float32),
        ],
        compiler_params=pltpu.CompilerParams(
            dimension_semantics=("parallel",)),
    )(embeddings, loss_signal, memory, importance)

    return out_mem, out_imp
